# baseline (speedup 1.0000x reference)
import jax
import jax.numpy as jnp
from jax.experimental import pallas as pl
from jax.experimental.pallas import tpu as pltpu

_LONG, _LAT = 128, 128
_N = _LONG * _LAT
_B = 64
_OUT = 256
_OFFSETS = (-1, 1, _LAT, -_LAT, _LAT - 1, _LAT + 1, -_LAT - 1, -_LAT + 1)
_LO, _HI = _LAT + 1, (_LONG - 1) * _LAT - 2
_NS = 4
_KC = _N // _NS


def _gnn_kernel(xf_ref, w_hbm, b_ref, out_ref, wv_ref, h1_ref, sems):
    for i in range(_NS):
        pltpu.make_async_copy(
            w_hbm.at[:, i * _KC:(i + 1) * _KC],
            wv_ref.at[:, i * _KC:(i + 1) * _KC],
            sems.at[i],
        ).start()

    idx = jax.lax.broadcasted_iota(jnp.int32, (1, _N), 1)
    m0 = ((idx >= _LO) & (idx <= _HI)).astype(jnp.float32)
    xv = xf_ref[...]
    h = xv
    for o in _OFFSETS:
        mo = ((idx + o >= _LO) & (idx + o <= _HI)).astype(jnp.float32)
        h = h + (m0 + mo) * pltpu.roll(xv, (-o) % _N, 1)
    h1_ref[...] = h

    acc = None
    for i in range(_NS):
        pltpu.make_async_copy(
            w_hbm.at[:, i * _KC:(i + 1) * _KC],
            wv_ref.at[:, i * _KC:(i + 1) * _KC],
            sems.at[i],
        ).wait()
        p = jax.lax.dot_general(
            h1_ref[:, i * _KC:(i + 1) * _KC],
            wv_ref[:, i * _KC:(i + 1) * _KC],
            (((1,), (1,)), ((), ())),
            preferred_element_type=jnp.float32)
        acc = p if acc is None else acc + p
    out_ref[...] = acc + b_ref[...]


def kernel(x, W, b):
    xf = x.reshape(_B, _N)
    b2 = b.reshape(1, _OUT)
    return pl.pallas_call(
        _gnn_kernel,
        in_specs=[
            pl.BlockSpec(memory_space=pltpu.MemorySpace.VMEM),
            pl.BlockSpec(memory_space=pltpu.MemorySpace.HBM),
            pl.BlockSpec(memory_space=pltpu.MemorySpace.VMEM),
        ],
        out_specs=pl.BlockSpec(memory_space=pltpu.MemorySpace.VMEM),
        out_shape=jax.ShapeDtypeStruct((_B, _OUT), jnp.float32),
        scratch_shapes=[
            pltpu.VMEM((_OUT, _N), jnp.float32),
            pltpu.VMEM((_B, _N), jnp.float32),
            pltpu.SemaphoreType.DMA((_NS,)),
        ],
    )(xf, W, b2)
